# baseline (speedup 1.0000x reference)
import functools

import jax
import jax.numpy as jnp
from jax import lax
from jax.experimental import pallas as pl
from jax.experimental.pallas import tpu as pltpu
from jax.experimental.pallas import tpu_sc as plsc

N_CLASSES = 1000000
EMBED_SIZE = 64
BATCH = 16384

_NC = 2
_NS = 16
_NW = _NC * _NS
_B_PER_W = BATCH // _NW
_CHUNK = 128
_N_CHUNKS = _B_PER_W // _CHUNK


def _make_kernel():
    mesh = plsc.VectorSubcoreMesh(core_axis_name="c", subcore_axis_name="s")

    @functools.partial(
        pl.kernel,
        mesh=mesh,
        out_type=jax.ShapeDtypeStruct((BATCH, EMBED_SIZE), jnp.float32),
        compiler_params=pltpu.CompilerParams(use_tc_tiling_on_sc=False),
        scratch_types=[
            pltpu.VMEM((_B_PER_W,), jnp.int32),
            pltpu.VMEM((_B_PER_W, EMBED_SIZE), jnp.float32),
            pltpu.SemaphoreType.DMA,
        ],
    )
    def embed(labels_hbm, table_hbm, out_hbm, idx_v, rows_v, sem):
        wid = lax.axis_index("s") * _NC + lax.axis_index("c")
        base = wid * _B_PER_W
        pltpu.sync_copy(labels_hbm.at[pl.ds(base, _B_PER_W)], idx_v)
        copies = []
        for j in range(_N_CHUNKS):
            copies.append(
                pltpu.async_copy(
                    table_hbm.at[idx_v.at[pl.ds(j * _CHUNK, _CHUNK)]],
                    rows_v.at[pl.ds(j * _CHUNK, _CHUNK)],
                    sem,
                )
            )
        for c in copies:
            c.wait()
        pltpu.sync_copy(rows_v, out_hbm.at[pl.ds(base, _B_PER_W)])

    return embed


_embed = jax.jit(_make_kernel())


def kernel(labels, table):
    return _embed(labels, table)


# speedup vs baseline: 1.7327x; 1.7327x over previous
import functools

import jax
import jax.numpy as jnp
from jax import lax
from jax.experimental import pallas as pl
from jax.experimental.pallas import tpu as pltpu
from jax.experimental.pallas import tpu_sc as plsc

N_CLASSES = 1000000
EMBED_SIZE = 64
BATCH = 16384

_NC = 2
_NS = 16
_NW = _NC * _NS
_B_PER_W = BATCH // _NW


def _make_kernel():
    mesh = plsc.VectorSubcoreMesh(core_axis_name="c", subcore_axis_name="s")

    @functools.partial(
        pl.kernel,
        mesh=mesh,
        out_type=jax.ShapeDtypeStruct((BATCH, EMBED_SIZE), jnp.float32),
        scratch_types=[
            pltpu.VMEM((_B_PER_W,), jnp.int32),
            pltpu.VMEM((_B_PER_W, EMBED_SIZE), jnp.float32),
            pltpu.SemaphoreType.DMA,
        ],
    )
    def embed(labels_hbm, table_hbm, out_hbm, idx_v, rows_v, sem):
        wid = lax.axis_index("s") * _NC + lax.axis_index("c")
        base = wid * _B_PER_W
        pltpu.sync_copy(labels_hbm.at[pl.ds(base, _B_PER_W)], idx_v)

        def body(g, _):
            v = idx_v[pl.ds(g * 16, 16)]
            for k in range(16):
                pltpu.async_copy(table_hbm.at[v[k]], rows_v.at[g * 16 + k], sem)
            return ()

        lax.fori_loop(0, _B_PER_W // 16, body, ())
        pltpu.make_async_copy(out_hbm.at[pl.ds(0, _B_PER_W)], rows_v, sem).wait()
        pltpu.sync_copy(rows_v, out_hbm.at[pl.ds(base, _B_PER_W)])

    return embed


_embed = jax.jit(_make_kernel())


def kernel(labels, table):
    return _embed(labels, table)


# speedup vs baseline: 2.5173x; 1.4528x over previous
import functools

import jax
import jax.numpy as jnp
from jax import lax
from jax.experimental import pallas as pl
from jax.experimental.pallas import tpu as pltpu
from jax.experimental.pallas import tpu_sc as plsc

N_CLASSES = 1000000
EMBED_SIZE = 64
BATCH = 16384

_NC = 2
_NS = 16
_NW = _NC * _NS
_B_PER_W = BATCH // _NW
_CH = 4
_N_CHUNKS = _B_PER_W // _CH


def _make_kernel():
    mesh = plsc.VectorSubcoreMesh(core_axis_name="c", subcore_axis_name="s")

    @functools.partial(
        pl.kernel,
        mesh=mesh,
        out_type=jax.ShapeDtypeStruct((EMBED_SIZE, BATCH), jnp.float32),
        compiler_params=pltpu.CompilerParams(needs_layout_passes=False),
        scratch_types=[
            pltpu.VMEM((_B_PER_W + 16,), jnp.int32),
            pltpu.VMEM((2, _CH * EMBED_SIZE, 128), jnp.float32),
            pltpu.VMEM((EMBED_SIZE, _B_PER_W), jnp.float32),
            pltpu.SemaphoreType.DMA,
            pltpu.SemaphoreType.DMA,
        ],
    )
    def embed(labels_hbm, table_t_hbm, out_t_hbm, idx_v, blk_v, cols_v, sem0, sem1):
        wid = lax.axis_index("s") * _NC + lax.axis_index("c")
        base = wid * _B_PER_W
        pltpu.sync_copy(labels_hbm.at[pl.ds(base, _B_PER_W)], idx_v.at[pl.ds(0, _B_PER_W)])

        sems = (sem0, sem1)
        iota16 = lax.iota(jnp.int32, 16)

        def fire(g, buf, sem):
            v = idx_v[pl.ds(g * _CH, 16)]
            for k in range(_CH):
                grp = v[k] >> 7
                start = pl.multiple_of(grp * 128, 128)
                pltpu.async_copy(
                    table_t_hbm.at[:, pl.ds(start, 128)],
                    blk_v.at[buf, pl.ds(k * EMBED_SIZE, EMBED_SIZE), :],
                    sem,
                )

        def drain(buf, sem):
            for k in range(_CH):
                pltpu.make_async_copy(
                    table_t_hbm.at[:, pl.ds(0, 128)],
                    blk_v.at[buf, pl.ds(k * EMBED_SIZE, EMBED_SIZE), :],
                    sem,
                ).wait()

        def extract(g, buf):
            v = idx_v[pl.ds(g * _CH, 16)]
            for k in range(_CH):
                col = v[k] & 127
                colv = jnp.full((16,), col, jnp.int32)
                for e0 in range(0, EMBED_SIZE, 16):
                    rows = jnp.full((16,), k * EMBED_SIZE + e0, jnp.int32) + iota16
                    vals = plsc.load_gather(blk_v.at[buf], [rows, colv])
                    orows = jnp.full((16,), e0, jnp.int32) + iota16
                    ocols = jnp.full((16,), g * _CH + k, jnp.int32)
                    plsc.store_scatter(cols_v, [orows, ocols], vals)

        def body(t, _):
            g0 = t * 2
            @pl.when(g0 + 1 < _N_CHUNKS)
            def _():
                fire(g0 + 1, 1, sem1)

            drain(0, sem0)
            extract(g0, 0)

            @pl.when(g0 + 2 < _N_CHUNKS)
            def _():
                fire(g0 + 2, 0, sem0)

            @pl.when(g0 + 1 < _N_CHUNKS)
            def _():
                drain(1, sem1)
                extract(g0 + 1, 1)

            return ()

        fire(0, 0, sem0)
        lax.fori_loop(0, (_N_CHUNKS + 1) // 2, body, ())
        pltpu.sync_copy(cols_v, out_t_hbm.at[:, pl.ds(base, _B_PER_W)])

    return embed


_embed = jax.jit(_make_kernel())


def kernel(labels, table):
    out_t = _embed(labels, table.T)
    return out_t.T


# speedup vs baseline: 2.7775x; 1.1034x over previous
import functools

import jax
import jax.numpy as jnp
from jax import lax
from jax.experimental import pallas as pl
from jax.experimental.pallas import tpu as pltpu
from jax.experimental.pallas import tpu_sc as plsc

N_CLASSES = 1000000
EMBED_SIZE = 64
BATCH = 16384

_NC = 2
_NS = 16
_NW = _NC * _NS
_B_PER_W = BATCH // _NW
_CH = 4
_N_CHUNKS = _B_PER_W // _CH
_NSLOTS = 3


def _make_kernel():
    mesh = plsc.VectorSubcoreMesh(core_axis_name="c", subcore_axis_name="s")

    @functools.partial(
        pl.kernel,
        mesh=mesh,
        out_type=jax.ShapeDtypeStruct((EMBED_SIZE, BATCH), jnp.float32),
        compiler_params=pltpu.CompilerParams(needs_layout_passes=False),
        scratch_types=[
            pltpu.VMEM((_B_PER_W + 16,), jnp.int32),
            pltpu.VMEM((_NSLOTS, _CH * EMBED_SIZE, 128), jnp.float32),
            pltpu.VMEM((EMBED_SIZE, 128), jnp.float32),
            pltpu.SemaphoreType.DMA,
            pltpu.SemaphoreType.DMA,
            pltpu.SemaphoreType.DMA,
        ],
    )
    def embed(labels_hbm, table_t_hbm, out_t_hbm, idx_v, blk_v, colq_v, s0, s1, s2):
        wid = lax.axis_index("s") * _NC + lax.axis_index("c")
        base = wid * _B_PER_W
        pltpu.sync_copy(labels_hbm.at[pl.ds(base, _B_PER_W)], idx_v.at[pl.ds(0, _B_PER_W)])

        sems = (s0, s1, s2)
        iota16 = lax.iota(jnp.int32, 16)

        def fire(c, slot):
            v = idx_v[pl.ds(c * _CH, 16)]
            for k in range(_CH):
                grp = v[k] >> 7
                start = pl.multiple_of(grp * 128, 128)
                pltpu.async_copy(
                    table_t_hbm.at[:, pl.ds(start, 128)],
                    blk_v.at[slot, pl.ds(k * EMBED_SIZE, EMBED_SIZE), :],
                    sems[slot],
                )

        def drain(slot):
            for k in range(_CH):
                pltpu.make_async_copy(
                    table_t_hbm.at[:, pl.ds(0, 128)],
                    blk_v.at[slot, pl.ds(k * EMBED_SIZE, EMBED_SIZE), :],
                    sems[slot],
                ).wait()

        def extract(c, slot):
            v = idx_v[pl.ds(c * _CH, 16)]
            for k in range(_CH):
                col = v[k] & 127
                colv = jnp.full((16,), col, jnp.int32)
                p = c * _CH + k
                ocols = jnp.full((16,), p & 127, jnp.int32)
                for e0 in range(0, EMBED_SIZE, 16):
                    rows = jnp.full((16,), k * EMBED_SIZE + e0, jnp.int32) + iota16
                    vals = plsc.load_gather(blk_v.at[slot], [rows, colv])
                    orows = jnp.full((16,), e0, jnp.int32) + iota16
                    plsc.store_scatter(colq_v, [orows, ocols], vals)

        for b in range(_NSLOTS):
            fire(b, b)

        def body(t, _):
            for b in range(_NSLOTS):
                c = t * _NSLOTS + b

                @pl.when(c < _N_CHUNKS)
                def _():
                    drain(b)
                    extract(c, b)

                    @pl.when(c + _NSLOTS < _N_CHUNKS)
                    def _():
                        fire(c + _NSLOTS, b)

                    @pl.when((c & 31) == 31)
                    def _():
                        q = c >> 5
                        off = pl.multiple_of(base + q * 128, 128)
                        pltpu.sync_copy(colq_v, out_t_hbm.at[:, pl.ds(off, 128)])

            return ()

        lax.fori_loop(0, (_N_CHUNKS + _NSLOTS - 1) // _NSLOTS, body, ())

    return embed


_embed = jax.jit(_make_kernel())


def kernel(labels, table):
    out_t = _embed(labels, table.T)
    return out_t.T


# speedup vs baseline: 3.0365x; 1.0932x over previous
import functools

import jax
import jax.numpy as jnp
from jax import lax
from jax.experimental import pallas as pl
from jax.experimental.pallas import tpu as pltpu
from jax.experimental.pallas import tpu_sc as plsc

N_CLASSES = 1000000
EMBED_SIZE = 64
BATCH = 16384

_NC = 2
_NS = 16
_NW = _NC * _NS
_B_PER_W = BATCH // _NW
_NSLOTS = 14


def _make_kernel():
    mesh = plsc.VectorSubcoreMesh(core_axis_name="c", subcore_axis_name="s")

    @functools.partial(
        pl.kernel,
        mesh=mesh,
        out_type=jax.ShapeDtypeStruct((EMBED_SIZE, BATCH), jnp.float32),
        compiler_params=pltpu.CompilerParams(needs_layout_passes=False),
        scratch_types=[
            pltpu.VMEM((_B_PER_W + 16,), jnp.int32),
            pltpu.VMEM((_NSLOTS, EMBED_SIZE, 128), jnp.float32),
            pltpu.VMEM((EMBED_SIZE, 128), jnp.float32),
        ]
        + [pltpu.SemaphoreType.DMA] * _NSLOTS,
    )
    def embed(labels_hbm, table_t_hbm, out_t_hbm, idx_v, blk_v, colq_v, *sems):
        wid = lax.axis_index("s") * _NC + lax.axis_index("c")
        base = wid * _B_PER_W
        pltpu.sync_copy(labels_hbm.at[pl.ds(base, _B_PER_W)], idx_v.at[pl.ds(0, _B_PER_W)])

        iota16 = lax.iota(jnp.int32, 16)

        def fire(p, slot):
            v = idx_v[pl.ds(p, 16)]
            grp = v[0] >> 7
            start = pl.multiple_of(grp * 128, 128)
            pltpu.async_copy(
                table_t_hbm.at[:, pl.ds(start, 128)],
                blk_v.at[slot],
                sems[slot],
            )

        def drain(slot):
            pltpu.make_async_copy(
                table_t_hbm.at[:, pl.ds(0, 128)],
                blk_v.at[slot],
                sems[slot],
            ).wait()

        def extract(p, slot):
            v = idx_v[pl.ds(p, 16)]
            colv = jnp.full((16,), v[0] & 127, jnp.int32)
            ocols = jnp.full((16,), p & 127, jnp.int32)
            for e0 in range(0, EMBED_SIZE, 16):
                rows = jnp.full((16,), e0, jnp.int32) + iota16
                vals = plsc.load_gather(blk_v.at[slot], [rows, colv])
                plsc.store_scatter(colq_v, [rows, ocols], vals)

        for b in range(_NSLOTS):
            fire(b, b)

        def body(t, _):
            for b in range(_NSLOTS):
                p = t * _NSLOTS + b

                @pl.when(p < _B_PER_W)
                def _():
                    drain(b)
                    extract(p, b)

                    @pl.when(p + _NSLOTS < _B_PER_W)
                    def _():
                        fire(p + _NSLOTS, b)

                    @pl.when((p & 127) == 127)
                    def _():
                        q = p >> 7
                        off = pl.multiple_of(base + q * 128, 128)
                        pltpu.sync_copy(colq_v, out_t_hbm.at[:, pl.ds(off, 128)])

            return ()

        lax.fori_loop(0, (_B_PER_W + _NSLOTS - 1) // _NSLOTS, body, ())

    return embed


_embed = jax.jit(_make_kernel())


def kernel(labels, table):
    out_t = _embed(labels, table.T)
    return out_t.T
